# baseline (speedup 1.0000x reference)
import functools

import jax
import jax.numpy as jnp
from jax import lax
from jax.experimental import pallas as pl
from jax.experimental.pallas import tpu as pltpu
from jax.experimental.pallas import tpu_sc as plsc

_B, _N, _F, _M, _H = 4, 4096, 64, 64, 8
_MH = _M * _H
_TR = 512
_S = 2048
_NSC = _N - _S
_NW = 32
_RPW = _NSC // _NW
_TILE = 8
_NT = _RPW // _TILE
_NCH = _N // 16


def _phase1_body(a_ref, stats_ref):
    r = pl.program_id(1)
    a = a_ref[0]
    rowsum = jnp.sum(a, axis=1)
    csum = jnp.sum(a, axis=0)

    dblk = a_ref[0, :, pl.ds(r * _TR, _TR)]
    ii = lax.broadcasted_iota(jnp.int32, (_TR, _TR), 0)
    jj = lax.broadcasted_iota(jnp.int32, (_TR, _TR), 1)
    dg = jnp.sum(jnp.where(ii == jj, dblk, 0.0), axis=1)

    @pl.when(r == 0)
    def _():
        stats_ref[0] = jnp.zeros((_N, 8), jnp.float32)

    stats_ref[0, :, 0:1] += csum[:, None]
    stats_ref[0, pl.ds(r * _TR, _TR), 1:2] = rowsum[:, None]
    stats_ref[0, pl.ds(r * _TR, _TR), 2:3] = dg[:, None]


def _sc_pass_body(a_ref, rowpart_ref, colpart_ref, diag_ref,
                  abuf0, abuf1, colacc, rowloc, diagv, sem0, sem1):
    wid = lax.axis_index("c") * 16 + lax.axis_index("s")
    base_row = _S + wid * _RPW
    bufs = (abuf0, abuf1)
    sems = (sem0, sem1)
    steps = [(b, t) for b in range(_B) for t in range(_NT)]
    iota16 = lax.iota(jnp.int32, 16)
    zero = jnp.zeros((16,), jnp.float32)

    def start(step):
        b, t = steps[step]
        row0 = base_row + t * _TILE
        return pltpu.async_copy(a_ref.at[b, pl.ds(row0, _TILE), :],
                                bufs[step % 2], sems[step % 2])

    pending = start(0)
    for step, (b, t) in enumerate(steps):
        pending.wait()
        if step + 1 < len(steps):
            pending = start(step + 1)
        buf = bufs[step % 2]

        if t == 0:
            def body0(j, accs):
                off = j * 16
                vs = [buf[i, pl.ds(off, 16)] for i in range(_TILE)]
                colacc[pl.ds(off, 16)] = (((vs[0] + vs[1]) + (vs[2] + vs[3]))
                                          + ((vs[4] + vs[5]) + (vs[6] + vs[7])))
                return tuple(accs[i] + vs[i] for i in range(_TILE))
            accs = lax.fori_loop(0, _NCH, body0, (zero,) * _TILE)
        else:
            def body(j, accs):
                off = j * 16
                vs = [buf[i, pl.ds(off, 16)] for i in range(_TILE)]
                colacc[pl.ds(off, 16)] += (((vs[0] + vs[1]) + (vs[2] + vs[3]))
                                           + ((vs[4] + vs[5]) + (vs[6] + vs[7])))
                return tuple(accs[i] + vs[i] for i in range(_TILE))
            accs = lax.fori_loop(0, _NCH, body, (zero,) * _TILE)

        for i in range(_TILE):
            rowloc[t * _TILE + i, :] = accs[i]

        dcol0 = base_row + t * _TILE
        chunk0 = (t * _TILE) % 16
        col0 = dcol0 - chunk0
        dvec = zero
        for i in range(_TILE):
            v = buf[i, pl.ds(col0, 16)]
            dvec = dvec + jnp.where(iota16 == (chunk0 + i), v, 0.0)
        if t % 2 == 0:
            pair_acc = dvec
        else:
            diagv[pl.ds((t // 2) * 16, 16)] = pair_acc + dvec

        if t == _NT - 1:
            pltpu.sync_copy(rowloc, rowpart_ref.at[b, pl.ds(wid * _RPW, _RPW)])
            pltpu.sync_copy(colacc, colpart_ref.at[b, wid])
            pltpu.sync_copy(diagv, diag_ref.at[b, wid])


_sc_pass = functools.partial(
    pl.kernel,
    out_type=[
        jax.ShapeDtypeStruct((_B, _NSC, 16), jnp.float32),
        jax.ShapeDtypeStruct((_B, _NW, _N), jnp.float32),
        jax.ShapeDtypeStruct((_B, _NW, _RPW), jnp.float32),
    ],
    mesh=plsc.VectorSubcoreMesh(core_axis_name="c", subcore_axis_name="s"),
    scratch_types=[
        pltpu.VMEM((_TILE, _N), jnp.float32),
        pltpu.VMEM((_TILE, _N), jnp.float32),
        pltpu.VMEM((_N,), jnp.float32),
        pltpu.VMEM((_RPW, 16), jnp.float32),
        pltpu.VMEM((_RPW,), jnp.float32),
        pltpu.SemaphoreType.DMA,
        pltpu.SemaphoreType.DMA,
    ],
)(_sc_pass_body)


def _phase2_body(x_ref, stats_ref, rowpart_ref, colpart_ref, diag_ref,
                 wn_ref, w2n_ref, wa_ref, sel_ref, binv_ref, psi_ref):
    inv_n = 1.0 / _N
    x = x_ref[0]
    h1 = jnp.dot(x, wn_ref[...], preferred_element_type=jnp.float32)
    sumx = jnp.sum(x, axis=0, keepdims=True)
    base = jnp.dot(sumx, w2n_ref[...], preferred_element_type=jnp.float32)

    cl = (stats_ref[0, :, 0:1]
          + jnp.sum(colpart_ref[0], axis=0)[:, None])
    rw_sc = jnp.sum(rowpart_ref[0], axis=1)[:, None]
    rw = jnp.concatenate([stats_ref[0, :_S, 1:2], rw_sc], axis=0)
    dg = jnp.concatenate([stats_ref[0, :_S, 2:3], diag_ref[0]], axis=0)

    sum_diag = jnp.sum(dg)
    suma = jnp.sum(rw)
    wa = wa_ref[...]
    base = (base + (sum_diag * inv_n) * wa[1:2]
            + (suma * inv_n * inv_n) * wa[4:5] + wa[5:6])
    pernode = (dg * wa[0:1] + (rw * inv_n) * wa[2:3]
               + (cl * inv_n) * wa[3:4])
    hidden = jnp.maximum(h1 + pernode + base, 0.0)
    s = jnp.sum(hidden, axis=0, keepdims=True)
    psi = jnp.dot(s, sel_ref[...], preferred_element_type=jnp.float32) * inv_n
    psi_ref[0, 0, :] = psi[0] + binv_ref[0]


def kernel(X, A, W_eq, b_eq, W_inv, b_inv):
    n = float(_N)
    Wx = W_eq[:, :, :_F, :]
    wav = W_eq[:, :, _F, :]
    Wn = (Wx[:, 0] + (Wx[:, 2] + Wx[:, 3]) * (1.0 / n))
    Wn = jnp.transpose(Wn, (1, 0, 2)).reshape(_F, _MH)
    W2n = (Wx[:, 1] * (1.0 / n) + Wx[:, 4] * (1.0 / (n * n)))
    W2n = jnp.transpose(W2n, (1, 0, 2)).reshape(_F, _MH)
    wa_rows = [wav[:, p].reshape(_MH) for p in range(5)]
    wa_pack = jnp.stack(wa_rows + [b_eq.reshape(_MH),
                                   jnp.zeros((_MH,), jnp.float32),
                                   jnp.zeros((_MH,), jnp.float32)])
    mh_ids = jnp.arange(_MH, dtype=jnp.int32) // _H
    sel = jnp.where(mh_ids[:, None] == jnp.arange(_M, dtype=jnp.int32)[None, :],
                    W_inv.reshape(_MH)[:, None], 0.0)

    rowpart, colpart, diag = _sc_pass(A)
    diag = diag.reshape(_B, _NSC, 1)

    stats = pl.pallas_call(
        _phase1_body,
        grid=(_B, _S // _TR),
        in_specs=[pl.BlockSpec((1, _TR, _N), lambda b, r: (b, r, 0))],
        out_specs=pl.BlockSpec((1, _N, 8), lambda b, r: (b, 0, 0)),
        out_shape=jax.ShapeDtypeStruct((_B, _N, 8), jnp.float32),
    )(A)

    psi = pl.pallas_call(
        _phase2_body,
        grid=(_B,),
        in_specs=[
            pl.BlockSpec((1, _N, _F), lambda b: (b, 0, 0)),
            pl.BlockSpec((1, _N, 8), lambda b: (b, 0, 0)),
            pl.BlockSpec((1, _NSC, 16), lambda b: (b, 0, 0)),
            pl.BlockSpec((1, _NW, _N), lambda b: (b, 0, 0)),
            pl.BlockSpec((1, _NSC, 1), lambda b: (b, 0, 0)),
            pl.BlockSpec((_F, _MH), lambda b: (0, 0)),
            pl.BlockSpec((_F, _MH), lambda b: (0, 0)),
            pl.BlockSpec((8, _MH), lambda b: (0, 0)),
            pl.BlockSpec((_MH, _M), lambda b: (0, 0)),
            pl.BlockSpec((1, _M), lambda b: (0, 0)),
        ],
        out_specs=pl.BlockSpec((1, 1, _M), lambda b: (b, 0, 0)),
        out_shape=jax.ShapeDtypeStruct((_B, 1, _M), jnp.float32),
    )(X, stats, rowpart, colpart, diag, Wn, W2n, wa_pack, sel,
      b_inv.reshape(1, _M))
    return psi.reshape(_B, _M)
